# baseline (speedup 1.0000x reference)
import functools

import jax
import jax.numpy as jnp
import numpy as np
from jax import lax
from jax.experimental import pallas as pl
from jax.experimental.pallas import tpu as pltpu
from jax.experimental.pallas import tpu_sc as plsc

_FIELD_DIMS = [1000] * 26
_NUM_FIELDS = len(_FIELD_DIMS)
_EMBED = 128
_BATCH = 16384
_FIELD_SIZE = 1000

_N = _BATCH * _NUM_FIELDS
_NW = 32
_CHUNK = 64
_LANES = 16

_CACHE_FIELDS = 10
_CACHE_ROWS = _CACHE_FIELDS * _BATCH // 16
_STAGE_ROWS = _CACHE_ROWS // 16
_CACHED_Q = _CACHE_FIELDS * _BATCH
_PER_CW = _CACHED_Q // 16
_PER_HW = (_N - _CACHED_Q) // 16

_mesh = plsc.VectorSubcoreMesh(core_axis_name="c", subcore_axis_name="s")


@functools.partial(
    pl.kernel,
    mesh=_mesh,
    out_type=jax.ShapeDtypeStruct((_NUM_FIELDS, _BATCH, _EMBED), jnp.float32),
    scratch_types=[
        pltpu.VMEM((_PER_HW,), jnp.int32),
        pltpu.VMEM((4, _CHUNK, _EMBED), jnp.float32),
        pltpu.VMEM_SHARED((_CACHE_ROWS, _EMBED), jnp.float32),
        pltpu.SemaphoreType.DMA,
        pltpu.SemaphoreType.DMA,
    ],
    compiler_params=pltpu.CompilerParams(use_tc_tiling_on_sc=True),
)
def _emb_lookup(x_hbm, table_hbm, out_hbm, idx_v, rows_v, cache_sp, gsem,
                wsem):
    sid = lax.axis_index("s")
    wid = sid * 2 + lax.axis_index("c")

    r0 = sid * _STAGE_ROWS
    pltpu.sync_copy(table_hbm.at[pl.ds(r0, _STAGE_ROWS)],
                    cache_sp.at[pl.ds(r0, _STAGE_ROWS)])
    plsc.subcore_barrier()

    def _run(q_start, per_w, src_ref):
        nchunk = per_w // _CHUNK
        pltpu.sync_copy(x_hbm.at[pl.ds(q_start, per_w)],
                        idx_v.at[pl.ds(0, per_w)])

        def _add_chunk(j):
            q0 = q_start + j * _CHUNK
            off = (q0 // _BATCH) * _FIELD_SIZE
            for v in range(_CHUNK // _LANES):
                sl = pl.ds(j * _CHUNK + v * _LANES, _LANES)
                idx_v[sl] = idx_v[sl] + off

        def _dst(j):
            q0 = q_start + j * _CHUNK
            return out_hbm.at[q0 // _BATCH].at[pl.ds(q0 % _BATCH, _CHUNK)]

        def _start_gather(j, b):
            pltpu.async_copy(src_ref.at[idx_v.at[pl.ds(j * _CHUNK, _CHUNK)]],
                             rows_v.at[b], gsem)

        def _wait_gather(j, b):
            pltpu.make_async_copy(
                src_ref.at[idx_v.at[pl.ds(j * _CHUNK, _CHUNK)]],
                rows_v.at[b], gsem).wait()

        def _start_write(j, b):
            pltpu.async_copy(rows_v.at[b], _dst(j), wsem)

        def _wait_write(j, b):
            pltpu.make_async_copy(rows_v.at[b], _dst(j), wsem).wait()

        for j in range(4):
            _add_chunk(j)
        _start_gather(0, 0)
        _start_gather(1, 1)
        for j in (0, 1):
            _wait_gather(j, j)
            _start_write(j, j)
            _add_chunk(j + 4)
            _start_gather(j + 2, j + 2)

        def _steady(s, carry):
            for k in range(4):
                j = 2 + s * 4 + k
                b = (2 + k) % 4
                _wait_gather(j, b)
                _start_write(j, b)
                _wait_write(j - 2, (b + 2) % 4)
                _add_chunk(j + 4)
                _start_gather(j + 2, (b + 2) % 4)
            return carry

        lax.fori_loop(0, (nchunk - 8) // 4, _steady, 0)

        for j in range(nchunk - 6, nchunk):
            b = j % 4
            _wait_gather(j, b)
            _start_write(j, b)
            _wait_write(j - 2, (b + 2) % 4)
            if j + 4 < nchunk:
                _add_chunk(j + 4)
            if j + 2 < nchunk:
                _start_gather(j + 2, (b + 2) % 4)
        for j in range(nchunk - 2, nchunk):
            _wait_write(j, j % 4)

    @pl.when(wid < 16)
    def _cached():
        _run(wid * _PER_CW, _PER_CW, cache_sp)

    @pl.when(wid >= 16)
    def _uncached():
        _run(_CACHED_Q + (wid - 16) * _PER_HW, _PER_HW, table_hbm)


def kernel(x, table):
    x_fmajor = x.astype(jnp.int32).T.reshape(_N)
    out = _emb_lookup(x_fmajor, table)
    return out.transpose(1, 0, 2)


# speedup vs baseline: 1.1790x; 1.1790x over previous
import functools

import jax
import jax.numpy as jnp
import numpy as np
from jax import lax
from jax.experimental import pallas as pl
from jax.experimental.pallas import tpu as pltpu
from jax.experimental.pallas import tpu_sc as plsc

_FIELD_DIMS = [1000] * 26
_NUM_FIELDS = len(_FIELD_DIMS)
_EMBED = 128
_BATCH = 16384
_FIELD_SIZE = 1000

_N = _BATCH * _NUM_FIELDS
_NW = 32
_PER_W = _N // _NW
_CHUNK = 128
_NCHUNK = _PER_W // _CHUNK
_LANES = 16

_mesh = plsc.VectorSubcoreMesh(core_axis_name="c", subcore_axis_name="s")


@functools.partial(
    pl.kernel,
    mesh=_mesh,
    out_type=jax.ShapeDtypeStruct((_NUM_FIELDS, _BATCH, _EMBED), jnp.float32),
    scratch_types=[
        pltpu.VMEM((_PER_W,), jnp.int32),
        pltpu.VMEM((6, _CHUNK, _EMBED), jnp.float32),
        pltpu.SemaphoreType.DMA,
        pltpu.SemaphoreType.DMA,
    ],
    compiler_params=pltpu.CompilerParams(use_tc_tiling_on_sc=True),
)
def _emb_lookup(x_hbm, table_hbm, out_hbm, idx_v, rows_v, gsem, wsem):
    wid = lax.axis_index("s") * 2 + lax.axis_index("c")
    q_base = wid * _PER_W

    pltpu.sync_copy(x_hbm.at[pl.ds(q_base, _PER_W)], idx_v)

    def _add_chunk(j):
        off = ((q_base + j * _CHUNK) // _BATCH) * _FIELD_SIZE
        for v in range(_CHUNK // _LANES):
            sl = pl.ds(j * _CHUNK + v * _LANES, _LANES)
            idx_v[sl] = idx_v[sl] + off

    def _dst(j):
        q0 = q_base + j * _CHUNK
        return out_hbm.at[q0 // _BATCH].at[pl.ds(q0 % _BATCH, _CHUNK)]

    def _start_gather(j, b):
        pltpu.async_copy(table_hbm.at[idx_v.at[pl.ds(j * _CHUNK, _CHUNK)]],
                         rows_v.at[b], gsem)

    def _wait_gather(j, b):
        pltpu.make_async_copy(
            table_hbm.at[idx_v.at[pl.ds(j * _CHUNK, _CHUNK)]],
            rows_v.at[b], gsem).wait()

    def _start_write(j, b):
        pltpu.async_copy(rows_v.at[b], _dst(j), wsem)

    def _wait_write(j, b):
        pltpu.make_async_copy(rows_v.at[b], _dst(j), wsem).wait()

    for j in range(6):
        _add_chunk(j)
    for j in (0, 1, 2):
        _start_gather(j, j)
    for j in (0, 1, 2):
        _wait_gather(j, j)
        _start_write(j, j)
        _add_chunk(j + 6)
        _start_gather(j + 3, j + 3)

    def _steady(s, carry):
        for k in range(6):
            j = 3 + s * 6 + k
            b = (3 + k) % 6
            _wait_gather(j, b)
            _start_write(j, b)
            _wait_write(j - 3, (b + 3) % 6)
            _add_chunk(j + 6)
            _start_gather(j + 3, (b + 3) % 6)
        return carry

    lax.fori_loop(0, (_NCHUNK - 14) // 6, _steady, 0)

    for j in range(_NCHUNK - 11, _NCHUNK):
        b = j % 6
        _wait_gather(j, b)
        _start_write(j, b)
        _wait_write(j - 3, (b + 3) % 6)
        if j + 6 < _NCHUNK:
            _add_chunk(j + 6)
        if j + 3 < _NCHUNK:
            _start_gather(j + 3, (b + 3) % 6)
    for j in range(_NCHUNK - 3, _NCHUNK):
        _wait_write(j, j % 6)


def kernel(x, table):
    x_fmajor = x.astype(jnp.int32).T.reshape(_N)
    out = _emb_lookup(x_fmajor, table)
    return out.transpose(1, 0, 2)


# speedup vs baseline: 1.1809x; 1.0016x over previous
import functools

import jax
import jax.numpy as jnp
from jax import lax
from jax.experimental import pallas as pl
from jax.experimental.pallas import tpu as pltpu
from jax.experimental.pallas import tpu_sc as plsc

_FIELD_DIMS = [1000] * 26
_NUM_FIELDS = len(_FIELD_DIMS)
_EMBED = 128
_BATCH = 16384
_FIELD_SIZE = _FIELD_DIMS[0]

_N = _BATCH * _NUM_FIELDS
_NW = 32
_PER_W = _N // _NW
_CHUNK = 128
_NCHUNK = _PER_W // _CHUNK
_LANES = 16

_mesh = plsc.VectorSubcoreMesh(core_axis_name="c", subcore_axis_name="s")


@functools.partial(
    pl.kernel,
    mesh=_mesh,
    out_type=jax.ShapeDtypeStruct((_NUM_FIELDS, _BATCH, _EMBED), jnp.float32),
    scratch_types=[
        pltpu.VMEM((_PER_W,), jnp.int32),
        pltpu.VMEM((6, _CHUNK, _EMBED), jnp.float32),
        pltpu.SemaphoreType.DMA,
        pltpu.SemaphoreType.DMA,
    ],
    compiler_params=pltpu.CompilerParams(use_tc_tiling_on_sc=True),
)
def _emb_lookup(x_hbm, table_hbm, out_hbm, idx_v, rows_v, gsem, wsem):
    wid = lax.axis_index("s") * 2 + lax.axis_index("c")
    q_base = wid * _PER_W

    pltpu.sync_copy(x_hbm.at[pl.ds(q_base, _PER_W)], idx_v)

    def _add_chunk(j):
        off = ((q_base + j * _CHUNK) // _BATCH) * _FIELD_SIZE
        for v in range(_CHUNK // _LANES):
            sl = pl.ds(j * _CHUNK + v * _LANES, _LANES)
            idx_v[sl] = idx_v[sl] + off

    def _dst(j):
        q0 = q_base + j * _CHUNK
        return out_hbm.at[q0 // _BATCH].at[pl.ds(q0 % _BATCH, _CHUNK)]

    def _start_gather(j, b):
        pltpu.async_copy(table_hbm.at[idx_v.at[pl.ds(j * _CHUNK, _CHUNK)]],
                         rows_v.at[b], gsem)

    def _wait_gather(j, b):
        pltpu.make_async_copy(
            table_hbm.at[idx_v.at[pl.ds(j * _CHUNK, _CHUNK)]],
            rows_v.at[b], gsem).wait()

    def _start_write(j, b):
        pltpu.async_copy(rows_v.at[b], _dst(j), wsem)

    def _wait_write(j, b):
        pltpu.make_async_copy(rows_v.at[b], _dst(j), wsem).wait()

    for j in range(6):
        _add_chunk(j)
    for j in (0, 1, 2):
        _start_gather(j, j)
    for j in (0, 1, 2):
        _wait_gather(j, j)
        _start_write(j, j)
        _add_chunk(j + 6)
        _start_gather(j + 3, j + 3)

    def _steady(s, carry):
        for k in range(6):
            j = 3 + s * 6 + k
            b = (3 + k) % 6
            _wait_gather(j, b)
            _start_write(j, b)
            _wait_write(j - 3, (b + 3) % 6)
            _add_chunk(j + 6)
            _start_gather(j + 3, (b + 3) % 6)
        return carry

    lax.fori_loop(0, (_NCHUNK - 14) // 6, _steady, 0)

    for j in range(_NCHUNK - 11, _NCHUNK):
        b = j % 6
        _wait_gather(j, b)
        _start_write(j, b)
        _wait_write(j - 3, (b + 3) % 6)
        if j + 6 < _NCHUNK:
            _add_chunk(j + 6)
        if j + 3 < _NCHUNK:
            _start_gather(j + 3, (b + 3) % 6)
    for j in range(_NCHUNK - 3, _NCHUNK):
        _wait_write(j, j % 6)


def kernel(x, table):
    x_fmajor = x.astype(jnp.int32).T.reshape(_N)
    out = _emb_lookup(x_fmajor, table)
    return out.transpose(1, 0, 2)
